# baseline (speedup 1.0000x reference)
import functools
import math

import jax
import jax.numpy as jnp
from jax import lax
from jax.experimental import pallas as pl
from jax.experimental.pallas import tpu as pltpu
from jax.experimental.pallas import tpu_sc as plsc

_NC = 2
_NS = 16
_NW = _NC * _NS
_CHUNK = 128


def _sc_gather(pid2d, cid2d, bid2d, pemb, cemb, bemb, n_pad, nchunk):
    dp = pemb.shape[1]
    dc = cemb.shape[1]
    db = bemb.shape[1]
    nw = nchunk * _CHUNK
    mesh = plsc.VectorSubcoreMesh(core_axis_name="c", subcore_axis_name="s")

    @functools.partial(
        pl.kernel,
        mesh=mesh,
        compiler_params=pltpu.CompilerParams(use_tc_tiling_on_sc=False),
        out_type=jax.ShapeDtypeStruct((n_pad, dp + dc + db), jnp.float32),
        scratch_types=[
            pltpu.VMEM((nw,), jnp.int32),
            pltpu.VMEM((nw,), jnp.int32),
            pltpu.VMEM((nw,), jnp.int32),
            pltpu.VMEM((nw, dp), jnp.float32),
            pltpu.VMEM((nw, dc), jnp.float32),
            pltpu.VMEM((nw, db), jnp.float32),
            pltpu.SemaphoreType.DMA,
        ],
    )
    def k(pid_h, cid_h, bid_h, pemb_h, cemb_h, bemb_h,
          x_h,
          idxp, idxc, idxb, rowsp, rowsc, rowsb, sem):
        wid = lax.axis_index("s") * _NC + lax.axis_index("c")
        base0 = wid * nw
        pltpu.sync_copy(pid_h.at[pl.ds(base0, nw)], idxp)
        pltpu.sync_copy(cid_h.at[pl.ds(base0, nw)], idxc)
        pltpu.sync_copy(bid_h.at[pl.ds(base0, nw)], idxb)

        copies = []
        for j in range(nchunk):
            off = j * _CHUNK
            copies.append(pltpu.async_copy(
                pemb_h.at[idxp.at[pl.ds(off, _CHUNK)]],
                rowsp.at[pl.ds(off, _CHUNK)], sem))
            copies.append(pltpu.async_copy(
                cemb_h.at[idxc.at[pl.ds(off, _CHUNK)]],
                rowsc.at[pl.ds(off, _CHUNK)], sem))
            copies.append(pltpu.async_copy(
                bemb_h.at[idxb.at[pl.ds(off, _CHUNK)]],
                rowsb.at[pl.ds(off, _CHUNK)], sem))
        for c in copies:
            c.wait()
        pltpu.sync_copy(rowsp, x_h.at[pl.ds(base0, nw), pl.ds(0, dp)])
        pltpu.sync_copy(rowsc, x_h.at[pl.ds(base0, nw), pl.ds(dp, dc)])
        pltpu.sync_copy(rowsb,
                        x_h.at[pl.ds(base0, nw), pl.ds(dp + dc, db)])

    return k(pid2d, cid2d, bid2d, pemb, cemb, bemb)


def _mlp_slab(x, ppt, w0a, w0s, b0, w1, b1, prev, n_out, n_rows, blk,
              row_off):
    dx = x.shape[1]
    dh = w0a.shape[1]
    do = w1.shape[1]
    grid = n_rows // blk

    transposed_pp = isinstance(ppt, tuple) is False

    def body(*refs):
        if transposed_pp:
            x_r, ppt_r, w0_r, w0s_r, b0_r, w1_r, b1_r = refs[:7]
        else:
            x_r, pr_r, po_r, w0_r, w0s_r, b0_r, w1_r, b1_r = refs[:8]
        o_r = refs[-1]
        h = jnp.dot(x_r[:].astype(jnp.bfloat16), w0_r[:],
                    preferred_element_type=jnp.float32)
        if transposed_pp:
            ppv = ppt_r[:]
            s_t = jnp.concatenate([ppv[0:1, :], jnp.log(ppv[1:2, :])],
                                  axis=0)
            h = h + jnp.dot(s_t.T, w0s_r[:],
                            preferred_element_type=jnp.float32)
        else:
            h = h + pr_r[:] * w0s_r[0:1, :] \
                + jnp.log(po_r[:]) * w0s_r[1:2, :]
        h = jnp.maximum(h + b0_r[:], 0.0)
        o_r[:] = jnp.dot(h.astype(jnp.bfloat16), w1_r[:],
                         preferred_element_type=jnp.float32) + b1_r[:]

    boff = row_off // blk
    if transposed_pp:
        pp_specs = [pl.BlockSpec((2, blk), lambda i, o=boff: (0, i + o))]
        pp_args = [ppt]
    else:
        pp_specs = [pl.BlockSpec((blk, 1), lambda i: (i, 0)),
                    pl.BlockSpec((blk, 1), lambda i: (i, 0))]
        pp_args = list(ppt)
    in_specs = [
        pl.BlockSpec((blk, dx), lambda i: (i, 0)),
        *pp_specs,
        pl.BlockSpec((dx, dh), lambda i: (0, 0)),
        pl.BlockSpec((2, dh), lambda i: (0, 0)),
        pl.BlockSpec((1, dh), lambda i: (0, 0)),
        pl.BlockSpec((dh, do), lambda i: (0, 0)),
        pl.BlockSpec((1, do), lambda i: (0, 0)),
    ]
    args = [x, *pp_args, w0a, w0s, b0, w1, b1]
    aliases = {}
    if prev is not None:
        in_specs.append(pl.BlockSpec(memory_space=pl.ANY))
        args.append(prev)
        aliases = {len(args) - 1: 0}

    return pl.pallas_call(
        body,
        grid=(grid,),
        in_specs=in_specs,
        out_specs=pl.BlockSpec((blk, do), lambda i, o=boff: (i + o, 0)),
        out_shape=jax.ShapeDtypeStruct((n_out, do), jnp.float32),
        input_output_aliases=aliases,
    )(*args)


def kernel(product_id, category_id, brand_id, price, popularity,
           product_emb, category_emb, brand_emb, W0, b0, W1, b1):
    n = product_id.shape[0]
    per_round = _NW * _CHUNK
    nchunk = -(-n // per_round)
    n_pad = nchunk * per_round
    pad = n_pad - n

    pid = jnp.pad(product_id.astype(jnp.int32), (0, pad))
    cid = jnp.pad(category_id.astype(jnp.int32), (0, pad))
    bid = jnp.pad(brand_id.astype(jnp.int32), (0, pad))

    ppt = jnp.stack([
        jnp.pad(price, (0, pad)),
        jnp.pad(popularity, (0, pad), constant_values=1.0),
    ], axis=0)

    dcat = product_emb.shape[1] + category_emb.shape[1] + brand_emb.shape[1]
    w0a = W0[:dcat, :].astype(jnp.bfloat16)
    w0s = W0[dcat:dcat + 2, :]
    w1b = W1.astype(jnp.bfloat16)

    n_slabs = 5 if nchunk % 5 == 0 else 1
    nc_s = nchunk // n_slabs
    rows_s = nc_s * per_round
    out = None
    for s in range(n_slabs):
        lo = s * rows_s
        x_s = _sc_gather(
            lax.slice(pid, (lo,), (lo + rows_s,)),
            lax.slice(cid, (lo,), (lo + rows_s,)),
            lax.slice(bid, (lo,), (lo + rows_s,)),
            product_emb, category_emb, brand_emb, rows_s, nc_s)
        n_rows = min(rows_s, n - lo)
        if n_rows % 1024 == 0:
            blk, scal = 1024, ppt
        else:
            g = math.gcd(n_rows, lo)
            blk = max(d for d in range(8, 1025, 8) if g % d == 0)
            scal = (lax.slice(price, (lo,), (lo + n_rows,))[:, None],
                    lax.slice(popularity, (lo,), (lo + n_rows,))[:, None])
        out = _mlp_slab(x_s, scal, w0a, w0s, b0[None, :], w1b,
                        b1[None, :], out, n, n_rows, blk, lo)
    return out


# speedup vs baseline: 1.1001x; 1.1001x over previous
import functools

import jax
import jax.numpy as jnp
from jax import lax
from jax.experimental import pallas as pl
from jax.experimental.pallas import tpu as pltpu
from jax.experimental.pallas import tpu_sc as plsc

_NC = 2
_NS = 16
_NW = _NC * _NS
_CHUNK = 128


def _sc_gather(pid2d, cid2d, bid2d, pemb, cemb, bemb, n_pad, nchunk):
    dp = pemb.shape[1]
    dc = cemb.shape[1]
    db = bemb.shape[1]
    nw = nchunk * _CHUNK
    mesh = plsc.VectorSubcoreMesh(core_axis_name="c", subcore_axis_name="s")

    @functools.partial(
        pl.kernel,
        mesh=mesh,
        compiler_params=pltpu.CompilerParams(use_tc_tiling_on_sc=False),
        out_type=jax.ShapeDtypeStruct((n_pad, dp + dc + db), jnp.float32),
        scratch_types=[
            pltpu.VMEM((nw,), jnp.int32),
            pltpu.VMEM((nw,), jnp.int32),
            pltpu.VMEM((nw,), jnp.int32),
            pltpu.VMEM((nw, dp), jnp.float32),
            pltpu.VMEM((nw, dc), jnp.float32),
            pltpu.VMEM((nw, db), jnp.float32),
            pltpu.SemaphoreType.DMA,
        ],
    )
    def k(pid_h, cid_h, bid_h, pemb_h, cemb_h, bemb_h,
          x_h,
          idxp, idxc, idxb, rowsp, rowsc, rowsb, sem):
        wid = lax.axis_index("s") * _NC + lax.axis_index("c")
        base0 = wid * nw
        pltpu.sync_copy(pid_h.at[pl.ds(base0, nw)], idxp)
        pltpu.sync_copy(cid_h.at[pl.ds(base0, nw)], idxc)
        pltpu.sync_copy(bid_h.at[pl.ds(base0, nw)], idxb)

        copies = []
        for j in range(nchunk):
            off = j * _CHUNK
            copies.append(pltpu.async_copy(
                pemb_h.at[idxp.at[pl.ds(off, _CHUNK)]],
                rowsp.at[pl.ds(off, _CHUNK)], sem))
            copies.append(pltpu.async_copy(
                cemb_h.at[idxc.at[pl.ds(off, _CHUNK)]],
                rowsc.at[pl.ds(off, _CHUNK)], sem))
            copies.append(pltpu.async_copy(
                bemb_h.at[idxb.at[pl.ds(off, _CHUNK)]],
                rowsb.at[pl.ds(off, _CHUNK)], sem))
        for c in copies:
            c.wait()
        pltpu.sync_copy(rowsp, x_h.at[pl.ds(base0, nw), pl.ds(0, dp)])
        pltpu.sync_copy(rowsc, x_h.at[pl.ds(base0, nw), pl.ds(dp, dc)])
        pltpu.sync_copy(rowsb,
                        x_h.at[pl.ds(base0, nw), pl.ds(dp + dc, db)])

    return k(pid2d, cid2d, bid2d, pemb, cemb, bemb)


def _mlp_slab(x, ppt, w0a, w0s, b0, w1, b1, prev, n_out, n_rows, blk,
              row_off):
    dx = x.shape[1]
    dh = w0a.shape[1]
    do = w1.shape[1]
    grid = n_rows // blk

    def body(*refs):
        x_r, ppt_r, w0_r, w0s_r, b0_r, w1_r, b1_r = refs[:7]
        o_r = refs[-1]
        h = jnp.dot(x_r[:].astype(jnp.bfloat16), w0_r[:],
                    preferred_element_type=jnp.float32)
        ppv = ppt_r[:]
        s_t = jnp.concatenate([ppv[0:1, :], jnp.log(ppv[1:2, :])], axis=0)
        h = h + jnp.dot(s_t.T, w0s_r[:], preferred_element_type=jnp.float32)
        h = jnp.maximum(h + b0_r[:], 0.0)
        o_r[:] = jnp.dot(h.astype(jnp.bfloat16), w1_r[:],
                         preferred_element_type=jnp.float32) + b1_r[:]

    boff = row_off // blk
    in_specs = [
        pl.BlockSpec((blk, dx), lambda i: (i, 0)),
        pl.BlockSpec((2, blk), lambda i, o=boff: (0, i + o)),
        pl.BlockSpec((dx, dh), lambda i: (0, 0)),
        pl.BlockSpec((2, dh), lambda i: (0, 0)),
        pl.BlockSpec((1, dh), lambda i: (0, 0)),
        pl.BlockSpec((dh, do), lambda i: (0, 0)),
        pl.BlockSpec((1, do), lambda i: (0, 0)),
    ]
    args = [x, ppt, w0a, w0s, b0, w1, b1]
    aliases = {}
    if prev is not None:
        in_specs.append(pl.BlockSpec(memory_space=pl.ANY))
        args.append(prev)
        aliases = {7: 0}

    return pl.pallas_call(
        body,
        grid=(grid,),
        in_specs=in_specs,
        out_specs=pl.BlockSpec((blk, do), lambda i, o=boff: (i + o, 0)),
        out_shape=jax.ShapeDtypeStruct((n_out, do), jnp.float32),
        input_output_aliases=aliases,
    )(*args)


def kernel(product_id, category_id, brand_id, price, popularity,
           product_emb, category_emb, brand_emb, W0, b0, W1, b1):
    n = product_id.shape[0]
    per_round = _NW * _CHUNK
    nchunk = -(-n // per_round)
    n_pad = nchunk * per_round
    pad = n_pad - n

    pid = jnp.pad(product_id.astype(jnp.int32), (0, pad))
    cid = jnp.pad(category_id.astype(jnp.int32), (0, pad))
    bid = jnp.pad(brand_id.astype(jnp.int32), (0, pad))

    ppt = jnp.stack([
        jnp.pad(price, (0, pad)),
        jnp.pad(popularity, (0, pad), constant_values=1.0),
    ], axis=0)

    dcat = product_emb.shape[1] + category_emb.shape[1] + brand_emb.shape[1]
    w0a = W0[:dcat, :].astype(jnp.bfloat16)
    w0s = W0[dcat:dcat + 2, :]
    w1b = W1.astype(jnp.bfloat16)

    n_slabs = 5 if nchunk % 5 == 0 else 1
    nc_s = nchunk // n_slabs
    rows_s = nc_s * per_round
    out = None
    for s in range(n_slabs):
        lo = s * rows_s
        x_s = _sc_gather(
            lax.slice(pid, (lo,), (lo + rows_s,)),
            lax.slice(cid, (lo,), (lo + rows_s,)),
            lax.slice(bid, (lo,), (lo + rows_s,)),
            product_emb, category_emb, brand_emb, rows_s, nc_s)
        out = _mlp_slab(x_s, ppt, w0a, w0s, b0[None, :], w1b,
                        b1[None, :], out, n_pad, rows_s, 1024, lo)
    return out[:n]
